# baseline (speedup 1.0000x reference)
import functools

import jax
import jax.numpy as jnp
from jax import lax
from jax.experimental import pallas as pl
from jax.experimental.pallas import tpu as pltpu
from jax.experimental.pallas import tpu_sc as plsc

NC = 2
NS = 16
NW = NC * NS
L = 16
NB = 8
NSB = 16
BLK = 128
CHB = 8


def _rup(a, b):
    return (a + b - 1) // b * b


def _make_bucketize(E_PAD, N_PAD):
    ECH = E_PAD // NW
    RB = N_PAD // NB
    SB = N_PAD // NSB
    CAP = _rup(ECH + BLK, CHB * BLK)
    NBLK = CAP // BLK
    CAP2 = _rup(ECH + NSB * BLK, CHB * BLK)
    NBLK2 = CAP2 // BLK
    ITERS = ECH // L

    mesh = plsc.VectorSubcoreMesh(core_axis_name="c", subcore_axis_name="s")

    @functools.partial(
        pl.kernel,
        out_type=(
            jax.ShapeDtypeStruct((2, NB, NW, NBLK2, BLK), jnp.int32),
            jax.ShapeDtypeStruct((2, NB, NW, NBLK2, BLK), jnp.int32),
            jax.ShapeDtypeStruct((2, NB, NW, NBLK2, BLK), jnp.float32),
            jax.ShapeDtypeStruct((2 * NB * NW * 2 * L,), jnp.int32),
        ),
        mesh=mesh,
        scratch_types=[
            pltpu.VMEM((ECH,), jnp.int32),
            pltpu.VMEM((ECH,), jnp.int32),
            pltpu.VMEM((ECH,), jnp.float32),
            pltpu.VMEM((NBLK, BLK), jnp.int32),
            pltpu.VMEM((NBLK, BLK), jnp.int32),
            pltpu.VMEM((NBLK, BLK), jnp.float32),
            pltpu.VMEM((NBLK2, BLK), jnp.int32),
            pltpu.VMEM((NBLK2, BLK), jnp.int32),
            pltpu.VMEM((NBLK2, BLK), jnp.float32),
            pltpu.VMEM((2 * L,), jnp.int32),
        ],
        compiler_params=pltpu.CompilerParams(needs_layout_passes=False),
    )
    def bucketize(pd_h, ps_h, pv_h, nd_h, ns_h, nv_h,
                  odst, osrc, oval, ocnt,
                  d_v, s_v, v_v, sd_v, ss_v, sv_v, sd2, ss2, sv2, c_v):
        wid = lax.axis_index("c") * NS + lax.axis_index("s")
        base = wid * ECH
        ii = lax.iota(jnp.int32, L)
        zi = jnp.zeros((L,), jnp.int32)
        zf = jnp.zeros((L,), jnp.float32)
        for sgn, (dh, sh, vh) in enumerate(((pd_h, ps_h, pv_h),
                                            (nd_h, ns_h, nv_h))):
            pltpu.sync_copy(dh.at[pl.ds(base, ECH)], d_v)
            pltpu.sync_copy(sh.at[pl.ds(base, ECH)], s_v)
            pltpu.sync_copy(vh.at[pl.ds(base, ECH)], v_v)
            def bucket_body(b, _, sgn=sgn):
                lo = b * RB

                def body(i, cur):
                    d = d_v[pl.ds(i * L, L)]
                    s = s_v[pl.ds(i * L, L)]
                    v = v_v[pl.ds(i * L, L)]
                    m = (d >= lo) & (d < lo + RB)
                    inc = jnp.where(m, 1, 0).astype(jnp.int32)
                    pos = cur + plsc.cumsum(inc) - 1
                    pr = lax.shift_right_logical(pos, 7)
                    pc = lax.bitwise_and(pos, 127)
                    plsc.store_scatter(sd_v, [pr, pc], d - lo, mask=m)
                    plsc.store_scatter(ss_v, [pr, pc], s, mask=m)
                    plsc.store_scatter(sv_v, [pr, pc], v, mask=m)
                    return cur + plsc.all_reduce_population_count(m)

                cur = lax.fori_loop(0, ITERS, body, jnp.zeros((L,), jnp.int32))
                ncur = jnp.max(cur)
                it2 = lax.shift_right_logical(ncur + L - 1, 4)

                def seg_body(s, carry):
                    cur2, st_vec, nb_vec = carry
                    slo = s * SB
                    bstart = lax.shift_right_logical(jnp.max(cur2), 7)

                    def body2(i, cur2):
                        idx = i * L + ii
                        pr = lax.shift_right_logical(idx, 7)
                        pc = lax.bitwise_and(idx, 127)
                        d = plsc.load_gather(sd_v, [pr, pc])
                        sl_ = plsc.load_gather(ss_v, [pr, pc])
                        v = plsc.load_gather(sv_v, [pr, pc])
                        m = (idx < cur) & (sl_ >= slo) & (sl_ < slo + SB)
                        inc = jnp.where(m, 1, 0).astype(jnp.int32)
                        pos = cur2 + plsc.cumsum(inc) - 1
                        p2r = lax.shift_right_logical(pos, 7)
                        p2c = lax.bitwise_and(pos, 127)
                        plsc.store_scatter(sd2, [p2r, p2c], d, mask=m)
                        plsc.store_scatter(ss2, [p2r, p2c], sl_ - slo,
                                           mask=m)
                        plsc.store_scatter(sv2, [p2r, p2c], v, mask=m)
                        return cur2 + plsc.all_reduce_population_count(m)

                    cur2 = lax.fori_loop(0, it2, body2, cur2)
                    for q in range(BLK // L):
                        p = cur2 + q * L + ii
                        pr = lax.shift_right_logical(p, 7)
                        pc = lax.bitwise_and(p, 127)
                        plsc.store_scatter(sd2, [pr, pc], zi)
                        plsc.store_scatter(ss2, [pr, pc], zi)
                        plsc.store_scatter(sv2, [pr, pc], zf)
                    ncur2 = jnp.max(cur2)
                    nblk_s = lax.shift_right_logical(
                        ncur2 - bstart * BLK + BLK - 1, 7)
                    st_vec = jnp.where(ii == s, bstart, st_vec)
                    nb_vec = jnp.where(ii == s, nblk_s, nb_vec)
                    cur2 = jnp.broadcast_to(
                        lax.bitwise_and(ncur2 + BLK - 1, ~(BLK - 1)), (L,))
                    return (cur2, st_vec, nb_vec)

                zv = jnp.zeros((L,), jnp.int32)
                _, st_vec, nb_vec = lax.fori_loop(
                    0, NSB, seg_body, (zv, zv, zv))

                pltpu.sync_copy(sd2, odst.at[sgn, b, wid])
                pltpu.sync_copy(ss2, osrc.at[sgn, b, wid])
                pltpu.sync_copy(sv2, oval.at[sgn, b, wid])
                c_v[pl.ds(0, L)] = st_vec
                c_v[pl.ds(L, L)] = nb_vec
                pltpu.sync_copy(
                    c_v,
                    ocnt.at[pl.ds(((sgn * NB + b) * NW + wid) * 2 * L,
                                  2 * L)])
                return 0

            lax.fori_loop(0, NB, bucket_body, 0)

    return bucketize, NBLK2


def _make_hop(E_PAD, N_PAD, H, NBLK2):
    RB = N_PAD // NB
    SB = N_PAD // NSB
    TR = RB // NS
    SLR = 8 * (SB // (8 * 8))
    ZR = TR // 8
    assert ZR * 8 == TR and SLR % 8 == 0 and SLR * 8 == SB

    mesh = plsc.VectorSubcoreMesh(core_axis_name="c", subcore_axis_name="s")

    @functools.partial(
        pl.kernel,
        out_type=(
            jax.ShapeDtypeStruct((N_PAD, H), jnp.float32),
            jax.ShapeDtypeStruct((N_PAD, H), jnp.float32),
        ),
        mesh=mesh,
        scratch_types=[
            pltpu.VMEM_SHARED((RB, H), jnp.float32),
            pltpu.VMEM_SHARED((SB, H), jnp.float32),
            pltpu.VMEM((CHB, BLK), jnp.int32),
            pltpu.VMEM((CHB, BLK), jnp.int32),
            pltpu.VMEM((CHB, BLK), jnp.float32),
            pltpu.VMEM((BLK, H), jnp.float32),
            pltpu.VMEM((BLK, H), jnp.float32),
            pltpu.VMEM((ZR, H), jnp.float32),
            pltpu.VMEM((2 * 2 * 2 * L,), jnp.int32),
            pltpu.SemaphoreType.DMA,
            pltpu.SemaphoreType.DMA,
        ],
        compiler_params=pltpu.CompilerParams(needs_layout_passes=False),
    )
    def hop(z_h, odst, osrc, oval, ocnt, hp_h, hn_h,
            acc, slab, dl_v, sr_v, vl_v, rows0, rows1,
            zbuf, call_v, sem0, sem1):
        c = lax.axis_index("c")
        sid = lax.axis_index("s")
        ii = lax.iota(jnp.int32, L)
        zf = jnp.zeros((L,), jnp.float32)

        def zb(i, _):
            for k in range(H // L):
                zbuf[i, pl.ds(k * L, L)] = zf
            return 0

        lax.fori_loop(0, ZR, zb, 0)

        bufs = (rows0, rows1)
        sems = (sem0, sem1)

        def pass_body(bi, _):
            b = c * (NB // NC) + bi
            for sgn in range(2):
                off = sid * TR
                for zi in range(TR // ZR):
                    pltpu.sync_copy(zbuf, acc.at[pl.ds(off + zi * ZR, ZR)])
                for ri in range(2):
                    pltpu.sync_copy(
                        ocnt.at[pl.ds(
                            (((sgn * NB + b) * NW) + sid * 2 + ri) * 2 * L,
                            2 * L)],
                        call_v.at[pl.ds(ri * 2 * L, 2 * L)])
                plsc.subcore_barrier()

                def scale_add(rbuf, j):
                    def edge(e2, _):
                        for e in (e2 * 2, e2 * 2 + 1):
                            bv = plsc.load_gather(
                                vl_v, [jnp.broadcast_to(j, (L,)),
                                       jnp.broadcast_to(e, (L,))])
                            for k in range(H // L):
                                sl = pl.ds(k * L, L)
                                rbuf[e, sl] = rbuf[e, sl] * bv
                        return 0

                    lax.fori_loop(0, BLK // 2, edge, 0)
                    pltpu.sync_copy(rbuf, acc.at[dl_v.at[j]], add=True)

                def s_body(s, _, sgn=sgn, b=b):
                    @pl.when(sid < SB // SLR)
                    def _(s=s):
                        pltpu.sync_copy(
                            z_h.at[pl.ds(s * SB + sid * SLR, SLR)],
                            slab.at[pl.ds(sid * SLR, SLR)])

                    plsc.subcore_barrier()

                    def region_body(ri, _, sgn=sgn, b=b, s=s):
                        r = sid * 2 + ri
                        cmeta_s = call_v[pl.ds(ri * 2 * L, L)]
                        cmeta_n = call_v[pl.ds(ri * 2 * L + L, L)]
                        start = jnp.max(jnp.where(ii == s, cmeta_s, 0))
                        nblk = jnp.max(jnp.where(ii == s, cmeta_n, 0))
                        astart = lax.bitwise_and(start, ~(CHB - 1))
                        span = start + nblk - astart
                        nch = jnp.where(
                            nblk > 0,
                            lax.shift_right_logical(span + CHB - 1, 3), 0)

                        def chunk_body(ci, _, sgn=sgn, r=r, b=b,
                                       start=start, nblk=nblk,
                                       astart=astart):
                            cb = pl.multiple_of(astart + ci * CHB, CHB)
                            pltpu.sync_copy(
                                odst.at[sgn, b, r, pl.ds(cb, CHB)], dl_v)
                            pltpu.sync_copy(
                                osrc.at[sgn, b, r, pl.ds(cb, CHB)], sr_v)
                            pltpu.sync_copy(
                                oval.at[sgn, b, r, pl.ds(cb, CHB)], vl_v)
                            lb = jnp.maximum(0, start - cb)
                            ub = jnp.minimum(CHB, start + nblk - cb)

                            for j in range(2):
                                @pl.when(lb + j < ub)
                                def _(j=j):
                                    pltpu.async_copy(
                                        slab.at[sr_v.at[lb + j]],
                                        bufs[j], sems[j])

                            def pair_body(t, _):
                                for k in range(2):
                                    jk = lb + t * 2 + k

                                    @pl.when(jk < ub)
                                    def _(jk=jk, k=k):
                                        pltpu.make_async_copy(
                                            slab.at[sr_v.at[jk]],
                                            bufs[k], sems[k]).wait()
                                        scale_add(bufs[k], jk)

                                        @pl.when(jk + 2 < ub)
                                        def _(jk=jk, k=k):
                                            pltpu.async_copy(
                                                slab.at[sr_v.at[jk + 2]],
                                                bufs[k], sems[k])

                                return 0

                            lax.fori_loop(0, (ub - lb + 1) // 2,
                                          pair_body, 0)
                            return 0

                        lax.fori_loop(0, nch, chunk_body, 0)
                        return 0

                    lax.fori_loop(0, 2, region_body, 0)
                    plsc.subcore_barrier()
                    return 0

                lax.fori_loop(0, NSB, s_body, 0)

                h_h = hp_h if sgn == 0 else hn_h
                goff = b * RB + off
                pltpu.sync_copy(acc.at[pl.ds(off, TR)],
                                h_h.at[pl.ds(goff, TR)])
                plsc.subcore_barrier()
            return 0

        lax.fori_loop(0, NB // NC, pass_body, 0)

    return hop


def _tc_in(x, w, bvec, blk):
    N, D = x.shape
    H = w.shape[1]

    def body(x_ref, w_ref, b_ref, o_ref):
        o_ref[...] = jnp.tanh(
            jnp.dot(x_ref[...], w_ref[...],
                    preferred_element_type=jnp.float32) + b_ref[...])

    return pl.pallas_call(
        body,
        grid=(N // blk,),
        in_specs=[pl.BlockSpec((blk, D), lambda i: (i, 0)),
                  pl.BlockSpec((D, H), lambda i: (0, 0)),
                  pl.BlockSpec((1, H), lambda i: (0, 0))],
        out_specs=pl.BlockSpec((blk, H), lambda i: (i, 0)),
        out_shape=jax.ShapeDtypeStruct((N, H), jnp.float32),
    )(x, w, bvec.reshape(1, H))


def _tc_hop(hp, hn, wp, bp, wn, bn, wf, bf, blk):
    N, H = hp.shape

    def body(hp_ref, hn_ref, wp_ref, bp_ref, wn_ref, bn_ref,
             wft_ref, wfb_ref, bf_ref, o_ref):
        f32 = jnp.float32
        mp = jnp.dot(wp_ref[...], wft_ref[...], preferred_element_type=f32)
        mn = jnp.dot(wn_ref[...], wfb_ref[...], preferred_element_type=f32)
        cb = (jnp.dot(bp_ref[...], wft_ref[...], preferred_element_type=f32)
              + jnp.dot(bn_ref[...], wfb_ref[...], preferred_element_type=f32)
              + bf_ref[...])
        o_ref[...] = jnp.tanh(
            jnp.dot(hp_ref[...], mp, preferred_element_type=f32)
            + jnp.dot(hn_ref[...], mn, preferred_element_type=f32) + cb)

    full = lambda s: pl.BlockSpec(s, lambda i: tuple(0 for _ in s))
    return pl.pallas_call(
        body,
        grid=(N // blk,),
        in_specs=[pl.BlockSpec((blk, H), lambda i: (i, 0)),
                  pl.BlockSpec((blk, H), lambda i: (i, 0)),
                  full((H, H)), full((1, H)), full((H, H)), full((1, H)),
                  full((H, H)), full((H, H)), full((1, H))],
        out_specs=pl.BlockSpec((blk, H), lambda i: (i, 0)),
        out_shape=jax.ShapeDtypeStruct((N, H), jnp.float32),
    )(hp, hn, wp, bp.reshape(1, H), wn, bn.reshape(1, H),
      wf[:H], wf[H:], bf.reshape(1, H))


def kernel(x, A_pos_indices, A_pos_values, A_neg_indices, A_neg_values,
           W_in, b_in, W_pos, b_pos, W_neg, b_neg, W_fuse, b_fuse):
    N, D = x.shape
    H = W_in.shape[1]
    HOPS = W_pos.shape[0]
    E = A_pos_values.shape[0]

    ECH = _rup(-(-E // NW), L)
    E_PAD = ECH * NW
    assert H % L == 0
    RB = _rup(-(-N // NB), BLK)
    N_PAD = NB * RB

    def prep(ind, val):
        ind = ind.astype(jnp.int32)
        pad = E_PAD - E
        d = jnp.pad(ind[0], (0, pad))
        s = jnp.pad(ind[1], (0, pad))
        v = jnp.pad(val, (0, pad))
        return d, s, v

    pd, ps, pv = prep(A_pos_indices, A_pos_values)
    nd, ns, nv = prep(A_neg_indices, A_neg_values)

    bucketize, NBLK2 = _make_bucketize(E_PAD, N_PAD)
    odst, osrc, oval, ocnt = bucketize(pd, ps, pv, nd, ns, nv)

    hop_k = _make_hop(E_PAD, N_PAD, H, NBLK2)

    TCBLK = 896
    assert N_PAD % TCBLK == 0
    x_p = jnp.pad(x, ((0, N_PAD - N), (0, 0)))
    z = _tc_in(x_p, W_in, b_in, TCBLK)
    zs = [z]
    for hop in range(HOPS):
        hp, hn = hop_k(z, odst, osrc, oval, ocnt)
        z = _tc_hop(hp, hn, W_pos[hop], b_pos[hop], W_neg[hop], b_neg[hop],
                    W_fuse[hop], b_fuse[hop], TCBLK)
        zs.append(z)
    return jnp.stack(zs, axis=0)[:, :N]


# speedup vs baseline: 1.0299x; 1.0299x over previous
import functools

import jax
import jax.numpy as jnp
from jax import lax
from jax.experimental import pallas as pl
from jax.experimental.pallas import tpu as pltpu
from jax.experimental.pallas import tpu_sc as plsc

NC = 2
NS = 16
NW = NC * NS
L = 16
NB = 8
BLK = 128
CHB = 8


def _rup(a, b):
    return (a + b - 1) // b * b


def _make_bucketize(E_PAD, N):
    ECH = E_PAD // NW
    RB = _rup(-(-N // NB), BLK)
    CAP = _rup(ECH + BLK, CHB * BLK)
    NBLK = CAP // BLK
    ITERS = ECH // L

    mesh = plsc.VectorSubcoreMesh(core_axis_name="c", subcore_axis_name="s")

    @functools.partial(
        pl.kernel,
        out_type=(
            jax.ShapeDtypeStruct((2, NB, NW, NBLK, BLK), jnp.int32),
            jax.ShapeDtypeStruct((2, NB, NW, NBLK, BLK), jnp.int32),
            jax.ShapeDtypeStruct((2, NB, NW, NBLK, BLK), jnp.float32),
            jax.ShapeDtypeStruct((2 * NW * L,), jnp.int32),
        ),
        mesh=mesh,
        scratch_types=[
            pltpu.VMEM((ECH,), jnp.int32),
            pltpu.VMEM((ECH,), jnp.int32),
            pltpu.VMEM((ECH,), jnp.float32),
            pltpu.VMEM((NBLK, BLK), jnp.int32),
            pltpu.VMEM((NBLK, BLK), jnp.int32),
            pltpu.VMEM((NBLK, BLK), jnp.float32),
            pltpu.VMEM((L,), jnp.int32),
        ],
        compiler_params=pltpu.CompilerParams(needs_layout_passes=False),
    )
    def bucketize(pd_h, ps_h, pv_h, nd_h, ns_h, nv_h,
                  odst, osrc, oval, ocnt,
                  d_v, s_v, v_v, sd_v, ss_v, sv_v, c_v):
        wid = lax.axis_index("c") * NS + lax.axis_index("s")
        base = wid * ECH
        ii = lax.iota(jnp.int32, L)
        zi = jnp.zeros((L,), jnp.int32)
        zf = jnp.zeros((L,), jnp.float32)
        for sgn, (dh, sh, vh) in enumerate(((pd_h, ps_h, pv_h),
                                            (nd_h, ns_h, nv_h))):
            pltpu.sync_copy(dh.at[pl.ds(base, ECH)], d_v)
            pltpu.sync_copy(sh.at[pl.ds(base, ECH)], s_v)
            pltpu.sync_copy(vh.at[pl.ds(base, ECH)], v_v)
            cnts = jnp.zeros((L,), jnp.int32)
            for b in range(NB):
                lo = b * RB

                def body(i, cur, lo=lo):
                    d = d_v[pl.ds(i * L, L)]
                    s = s_v[pl.ds(i * L, L)]
                    v = v_v[pl.ds(i * L, L)]
                    m = (d >= lo) & (d < lo + RB)
                    inc = jnp.where(m, 1, 0).astype(jnp.int32)
                    pos = cur + plsc.cumsum(inc) - 1
                    pr = lax.shift_right_logical(pos, 7)
                    pc = lax.bitwise_and(pos, 127)
                    plsc.store_scatter(sd_v, [pr, pc], d - lo, mask=m)
                    plsc.store_scatter(ss_v, [pr, pc], s, mask=m)
                    plsc.store_scatter(sv_v, [pr, pc], v, mask=m)
                    return cur + plsc.all_reduce_population_count(m)

                cur = lax.fori_loop(0, ITERS, body, jnp.zeros((L,), jnp.int32))
                for q in range(BLK // L):
                    p = cur + q * L + ii
                    pr = lax.shift_right_logical(p, 7)
                    pc = lax.bitwise_and(p, 127)
                    plsc.store_scatter(sd_v, [pr, pc], zi)
                    plsc.store_scatter(ss_v, [pr, pc], zi)
                    plsc.store_scatter(sv_v, [pr, pc], zf)
                pltpu.sync_copy(sd_v, odst.at[sgn, b, wid])
                pltpu.sync_copy(ss_v, osrc.at[sgn, b, wid])
                pltpu.sync_copy(sv_v, oval.at[sgn, b, wid])
                cnts = jnp.where(ii == b, cur, cnts)
            c_v[...] = cnts
            pltpu.sync_copy(c_v, ocnt.at[pl.ds((sgn * NW + wid) * L, L)])

    return bucketize, CAP, NBLK, RB


def _make_hop(E_PAD, N, H):
    ECH = E_PAD // NW
    RB = _rup(-(-N // NB), BLK)
    CAP = _rup(ECH + BLK, CHB * BLK)
    NBLK = CAP // BLK
    TR = RB // NS
    ACC_R = RB
    TR_LAST = N - (NB - 1) * RB - (NS - 1) * TR
    assert 0 < TR_LAST <= TR and TR_LAST % 8 == 0 and TR % 8 == 0
    ZR = TR // 8
    assert ZR * 8 == TR

    mesh = plsc.VectorSubcoreMesh(core_axis_name="c", subcore_axis_name="s")

    @functools.partial(
        pl.kernel,
        out_type=(
            jax.ShapeDtypeStruct((N, H), jnp.float32),
            jax.ShapeDtypeStruct((N, H), jnp.float32),
        ),
        mesh=mesh,
        scratch_types=[
            pltpu.VMEM_SHARED((ACC_R, H), jnp.float32),
            pltpu.VMEM((CHB, BLK), jnp.int32),
            pltpu.VMEM((CHB, BLK), jnp.int32),
            pltpu.VMEM((CHB, BLK), jnp.float32),
            pltpu.VMEM((BLK, H), jnp.float32),
            pltpu.VMEM((BLK, H), jnp.float32),
            pltpu.VMEM((BLK, H), jnp.float32),
            pltpu.VMEM((BLK, H), jnp.float32),
            pltpu.VMEM((ZR, H), jnp.float32),
            pltpu.VMEM((L,), jnp.int32),
            pltpu.SemaphoreType.DMA,
            pltpu.SemaphoreType.DMA,
            pltpu.SemaphoreType.DMA,
            pltpu.SemaphoreType.DMA,
        ],
        compiler_params=pltpu.CompilerParams(needs_layout_passes=False),
    )
    def hop(z_h, odst, osrc, oval, ocnt, hp_h, hn_h,
            acc, dl_v, sr_v, vl_v, rows0, rows1, rows2, rows3, zbuf, c_v,
            sem0, sem1, sem2, sem3):
        c = lax.axis_index("c")
        sid = lax.axis_index("s")
        ii = lax.iota(jnp.int32, L)
        zf = jnp.zeros((L,), jnp.float32)

        def zb(i, _):
            for k in range(H // L):
                zbuf[i, pl.ds(k * L, L)] = zf
            return 0

        lax.fori_loop(0, ZR, zb, 0)

        for bi in range(NB // NC):
            b = c * (NB // NC) + bi
            for sgn in range(2):
                off = sid * TR
                for zi in range(TR // ZR):
                    pltpu.sync_copy(zbuf, acc.at[pl.ds(off + zi * ZR, ZR)])
                plsc.subcore_barrier()

                bufs = (rows0, rows1, rows2, rows3)
                sems = (sem0, sem1, sem2, sem3)

                def scale_add(rbuf, j):
                    def edge(e2, _):
                        for e in (e2 * 2, e2 * 2 + 1):
                            bv = plsc.load_gather(
                                vl_v, [jnp.broadcast_to(j, (L,)),
                                       jnp.broadcast_to(e, (L,))])
                            for k in range(H // L):
                                sl = pl.ds(k * L, L)
                                rbuf[e, sl] = rbuf[e, sl] * bv
                        return 0

                    lax.fori_loop(0, BLK // 2, edge, 0)
                    pltpu.sync_copy(rbuf, acc.at[dl_v.at[j]], add=True)

                def region_body(ri, _, sgn=sgn, b=b):
                    r = sid * 2 + ri
                    pltpu.sync_copy(
                        ocnt.at[pl.ds((sgn * NW + r) * L, L)], c_v)
                    n = jnp.max(jnp.where(ii == b, c_v[...], 0))
                    nblk = lax.shift_right_logical(n + BLK - 1, 7)
                    nch = lax.shift_right_logical(nblk + CHB - 1, 3)

                    def chunk_body(ci, _, sgn=sgn, r=r, b=b, nblk=nblk):
                        pltpu.sync_copy(
                            odst.at[sgn, b, r, pl.ds(ci * CHB, CHB)], dl_v)
                        pltpu.sync_copy(
                            osrc.at[sgn, b, r, pl.ds(ci * CHB, CHB)], sr_v)
                        pltpu.sync_copy(
                            oval.at[sgn, b, r, pl.ds(ci * CHB, CHB)], vl_v)
                        m = jnp.minimum(CHB, nblk - ci * CHB)

                        for j in range(4):
                            if j == 0:
                                pltpu.async_copy(z_h.at[sr_v.at[0]],
                                                 bufs[0], sems[0])
                            else:
                                @pl.when(j < m)
                                def _(j=j):
                                    pltpu.async_copy(z_h.at[sr_v.at[j]],
                                                     bufs[j], sems[j])

                        def quad_body(t, _):
                            for k in range(4):
                                jk = t * 4 + k

                                @pl.when(jk < m)
                                def _(jk=jk, k=k):
                                    pltpu.make_async_copy(
                                        z_h.at[sr_v.at[jk]],
                                        bufs[k], sems[k]).wait()
                                    scale_add(bufs[k], jk)

                                    @pl.when(jk + 4 < m)
                                    def _(jk=jk, k=k):
                                        pltpu.async_copy(
                                            z_h.at[sr_v.at[jk + 4]],
                                            bufs[k], sems[k])

                            return 0

                        lax.fori_loop(0, (m + 3) // 4, quad_body, 0)
                        return 0

                    lax.fori_loop(0, nch, chunk_body, 0)
                    return 0

                lax.fori_loop(0, 2, region_body, 0)

                plsc.subcore_barrier()
                h_h = hp_h if sgn == 0 else hn_h
                goff = b * RB + off
                short = (b == NB - 1) & (sid == NS - 1)

                @pl.when(jnp.logical_not(short))
                def _():
                    pltpu.sync_copy(acc.at[pl.ds(off, TR)],
                                    h_h.at[pl.ds(goff, TR)])

                @pl.when(short)
                def _():
                    pltpu.sync_copy(acc.at[pl.ds(off, TR_LAST)],
                                    h_h.at[pl.ds(goff, TR_LAST)])

                plsc.subcore_barrier()

    return hop


def _tc_in(x, w, bvec, blk):
    N, D = x.shape
    H = w.shape[1]

    def body(x_ref, w_ref, b_ref, o_ref):
        o_ref[...] = jnp.tanh(
            jnp.dot(x_ref[...], w_ref[...],
                    preferred_element_type=jnp.float32) + b_ref[...])

    return pl.pallas_call(
        body,
        grid=(N // blk,),
        in_specs=[pl.BlockSpec((blk, D), lambda i: (i, 0)),
                  pl.BlockSpec((D, H), lambda i: (0, 0)),
                  pl.BlockSpec((1, H), lambda i: (0, 0))],
        out_specs=pl.BlockSpec((blk, H), lambda i: (i, 0)),
        out_shape=jax.ShapeDtypeStruct((N, H), jnp.float32),
    )(x, w, bvec.reshape(1, H))


def _tc_hop(hp, hn, wp, bp, wn, bn, wf, bf, blk):
    N, H = hp.shape

    def body(hp_ref, hn_ref, wp_ref, bp_ref, wn_ref, bn_ref,
             wft_ref, wfb_ref, bf_ref, o_ref):
        f32 = jnp.float32
        mp = jnp.dot(wp_ref[...], wft_ref[...], preferred_element_type=f32)
        mn = jnp.dot(wn_ref[...], wfb_ref[...], preferred_element_type=f32)
        cb = (jnp.dot(bp_ref[...], wft_ref[...], preferred_element_type=f32)
              + jnp.dot(bn_ref[...], wfb_ref[...], preferred_element_type=f32)
              + bf_ref[...])
        o_ref[...] = jnp.tanh(
            jnp.dot(hp_ref[...], mp, preferred_element_type=f32)
            + jnp.dot(hn_ref[...], mn, preferred_element_type=f32) + cb)

    full = lambda s: pl.BlockSpec(s, lambda i: tuple(0 for _ in s))
    return pl.pallas_call(
        body,
        grid=(N // blk,),
        in_specs=[pl.BlockSpec((blk, H), lambda i: (i, 0)),
                  pl.BlockSpec((blk, H), lambda i: (i, 0)),
                  full((H, H)), full((1, H)), full((H, H)), full((1, H)),
                  full((H, H)), full((H, H)), full((1, H))],
        out_specs=pl.BlockSpec((blk, H), lambda i: (i, 0)),
        out_shape=jax.ShapeDtypeStruct((N, H), jnp.float32),
    )(hp, hn, wp, bp.reshape(1, H), wn, bn.reshape(1, H),
      wf[:H], wf[H:], bf.reshape(1, H))


def kernel(x, A_pos_indices, A_pos_values, A_neg_indices, A_neg_values,
           W_in, b_in, W_pos, b_pos, W_neg, b_neg, W_fuse, b_fuse):
    N, D = x.shape
    H = W_in.shape[1]
    HOPS = W_pos.shape[0]
    E = A_pos_values.shape[0]

    ECH = _rup(-(-E // NW), L)
    E_PAD = ECH * NW
    assert H % L == 0

    def prep(ind, val):
        ind = ind.astype(jnp.int32)
        pad = E_PAD - E
        d = jnp.pad(ind[0], (0, pad))
        s = jnp.pad(ind[1], (0, pad))
        v = jnp.pad(val, (0, pad))
        return d, s, v

    pd, ps, pv = prep(A_pos_indices, A_pos_values)
    nd, ns, nv = prep(A_neg_indices, A_neg_values)

    bucketize, _, _, _ = _make_bucketize(E_PAD, N)
    odst, osrc, oval, ocnt = bucketize(pd, ps, pv, nd, ns, nv)

    hop_k = _make_hop(E_PAD, N, H)

    TCBLK = 1000
    z = _tc_in(x, W_in, b_in, TCBLK)
    zs = [z]
    for hop in range(HOPS):
        hp, hn = hop_k(z, odst, osrc, oval, ocnt)
        z = _tc_hop(hp, hn, W_pos[hop], b_pos[hop], W_neg[hop], b_neg[hop],
                    W_fuse[hop], b_fuse[hop], TCBLK)
        zs.append(z)
    return jnp.stack(zs, axis=0)
